# baseline (speedup 1.0000x reference)
import functools

import jax
import jax.numpy as jnp
from jax import lax
from jax.experimental import pallas as pl
from jax.experimental.pallas import tpu as pltpu
from jax.experimental.pallas import tpu_sc as plsc

N = 10000
E = 160000
D = 256
HALF = D // 2

NC = 2
NS = 16
ET = E // NS
CB = 80
GC = 8
ETP = 10240
CK = ETP // CB
NG = CK // GC
NP = 10112
NR = NP // NS

@functools.cache
def _get_segsum_call():
    mesh = plsc.VectorSubcoreMesh(core_axis_name="c", subcore_axis_name="s",
                                  num_cores=NC, num_subcores=NS)
    return pl.kernel(
        _segsum_body,
        out_type=jax.ShapeDtypeStruct((NC, NP, HALF), jnp.float32),
        mesh=mesh,
        scratch_types=[
            pltpu.VMEM((4, GC, CB), jnp.int32),
            pltpu.VMEM((4, GC, CB), jnp.int32),
            pltpu.VMEM((4, CB, HALF), jnp.float32),
            pltpu.VMEM_SHARED((NP, HALF), jnp.float32),
            pltpu.SemaphoreType.DMA((4,)),
            pltpu.SemaphoreType.DMA((4,)),
            pltpu.SemaphoreType.DMA((4,)),
        ],
    )


def _segsum_body(x2_hbm, idx2_hbm, dst_hbm, out_hbm,
                 idx_g, dst_g, rows_v, acc, gsems, isems, dsems):
    c = lax.axis_index("c")
    s = lax.axis_index("s")

    def _zero_row(i, carry):
        for j in range(HALF // 16):
            rows_v[0, i, 16 * j:16 * (j + 1)] = jnp.zeros((16,), jnp.float32)
        return carry
    lax.fori_loop(0, CB, _zero_row, 0)
    for off in range(0, NR, CB):
        n = min(CB, NR - off)
        pltpu.sync_copy(rows_v.at[0, pl.ds(0, n)],
                        acc.at[pl.ds(s * NR + off, n)])
    plsc.subcore_barrier()

    def _group_fetch(g):
        pltpu.async_copy(idx2_hbm.at[c, s, g], idx_g.at[g % 4],
                         isems.at[g % 4])
        pltpu.async_copy(dst_hbm.at[s, g], dst_g.at[g % 4], dsems.at[g % 4])

    def _idx_wait(g):
        pltpu.make_async_copy(idx2_hbm.at[c, s, g], idx_g.at[g % 4],
                              isems.at[g % 4]).wait()

    def _dst_wait(g):
        pltpu.make_async_copy(dst_hbm.at[s, g], dst_g.at[g % 4],
                              dsems.at[g % 4]).wait()

    def _gather(k):
        pltpu.async_copy(x2_hbm.at[idx_g.at[(k // GC) % 4, k % GC]],
                         rows_v.at[k % 4], gsems.at[k % 4])

    def _gather_wait(k):
        pltpu.make_async_copy(x2_hbm.at[idx_g.at[(k // GC) % 4, k % GC]],
                              rows_v.at[k % 4], gsems.at[k % 4]).wait()

    for g in range(4):
        _group_fetch(g)
    _idx_wait(0)
    for p in range(3):
        _gather(p)

    def _chunk(k, carry):
        _gather_wait(k)

        @pl.when(jnp.logical_and(k + 3 < CK, (k + 3) % GC == 0))
        def _():
            _idx_wait((k + 3) // GC)

        @pl.when(k + 3 < CK)
        def _():
            _gather(k + 3)

        @pl.when(k % GC == 0)
        def _():
            _dst_wait(k // GC)

        pltpu.sync_copy(rows_v.at[k % 4],
                        acc.at[dst_g.at[(k // GC) % 4, k % GC]], add=True)

        @pl.when(jnp.logical_and(k % GC == GC - 1, k // GC + 4 < NG))
        def _():
            _group_fetch(k // GC + 4)

        return carry

    lax.fori_loop(0, CK, _chunk, 0)
    plsc.subcore_barrier()

    pltpu.sync_copy(acc.at[pl.ds(s * NR, NR)], out_hbm.at[c, pl.ds(s * NR, NR)])


def _mlp_body(add_residual, fuse_head, x_ref, agg_ref, Wa_ref, ba_ref,
              Wb_ref, bb_ref, g_ref, be_ref, *rest):
    if fuse_head:
        Wh1_ref, bh1_ref, Wh2_ref, bh2_ref, out_ref = rest
    else:
        (out_ref,) = rest
    xb = x_ref[...]
    h0 = xb + jnp.concatenate([agg_ref[0], agg_ref[1]], axis=-1)
    h = jnp.dot(h0, Wa_ref[...], preferred_element_type=jnp.float32)
    h = jnp.maximum(h + ba_ref[...], 0.0)
    h = jnp.dot(h, Wb_ref[...], preferred_element_type=jnp.float32) + bb_ref[...]
    mu = jnp.mean(h, axis=-1, keepdims=True)
    var = jnp.mean((h - mu) ** 2, axis=-1, keepdims=True)
    h = (h - mu) / jnp.sqrt(var + 1e-5) * g_ref[...] + be_ref[...]
    h = jnp.maximum(h, 0.0)
    if add_residual:
        h = h + xb
    if fuse_head:
        h = jnp.maximum(
            jnp.dot(h, Wh1_ref[...], preferred_element_type=jnp.float32)
            + bh1_ref[...], 0.0)
        h = jnp.dot(h, Wh2_ref[...], preferred_element_type=jnp.float32) \
            + bh2_ref[...]
    out_ref[...] = h


_TILE = 1000


def _mlp_call(x, agg, weights, add_residual, fuse_head):
    w_specs = []
    for w in weights:
        if w.ndim == 1:
            w = w.reshape(1, -1)
        w_specs.append((w, pl.BlockSpec(w.shape, lambda i: (0, 0))))
    return pl.pallas_call(
        functools.partial(_mlp_body, add_residual, fuse_head),
        grid=(N // _TILE,),
        in_specs=[
            pl.BlockSpec((_TILE, D), lambda i: (i, 0)),
            pl.BlockSpec((NC, _TILE, HALF), lambda i: (0, i, 0)),
        ] + [spec for _, spec in w_specs],
        out_specs=pl.BlockSpec((_TILE, D), lambda i: (i, 0)),
        out_shape=jax.ShapeDtypeStruct((N, D), jnp.float32),
    )(x, agg, *[w for w, _ in w_specs])


def kernel(x, edge_index, W0a, b0a, W0b, b0b, g0, be0, W1a, b1a, W1b, b1b,
           g1, be1, W2a, b2a, W2b, b2b, g2, be2, Wh1, bh1, Wh2, bh2):
    src = edge_index[0]
    dst = edge_index[1]
    idx2 = jnp.pad(jnp.stack([src * 2, src * 2 + 1]).reshape(NC, NS, ET),
                   ((0, 0), (0, 0), (0, ETP - ET))
                   ).reshape(NC, NS, NG, GC, CB)
    pad_dst = jnp.broadcast_to(
        N + jnp.arange(ETP - ET, dtype=jnp.int32) % (NP - N), (NS, ETP - ET))
    dstr = jnp.concatenate([dst.reshape(NS, ET), pad_dst],
                           axis=1).reshape(NS, NG, GC, CB)

    def seg(h):
        return _get_segsum_call()(h.reshape(2 * N, HALF), idx2, dstr)

    h = _mlp_call(x, seg(x), (W0a, b0a, W0b, b0b, g0, be0), False, False)
    h = _mlp_call(h, seg(h), (W1a, b1a, W1b, b1b, g1, be1), True, False)
    out = _mlp_call(h, seg(h),
                    (W2a, b2a, W2b, b2b, g2, be2, Wh1, bh1, Wh2, bh2),
                    True, True)
    return out


# speedup vs baseline: 2.7174x; 2.7174x over previous
import functools

import jax
import jax.numpy as jnp
from jax import lax
from jax.experimental import pallas as pl
from jax.experimental.pallas import tpu as pltpu
from jax.experimental.pallas import tpu_sc as plsc

N = 10000
E = 160000
D = 256
HALF = D // 2

NC = 2
NS = 16
ET = E // NS
CB = 80
ETP = 10000
CK = ETP // CB
NP = 10240
NR = NP // NS

@functools.cache
def _get_segsum_call():
    mesh = plsc.VectorSubcoreMesh(core_axis_name="c", subcore_axis_name="s",
                                  num_cores=NC, num_subcores=NS)
    return pl.kernel(
        _segsum_body,
        out_type=jax.ShapeDtypeStruct((NC, NP, HALF), jnp.float32),
        mesh=mesh,
        scratch_types=[
            pltpu.VMEM((ETP,), jnp.int32),
            pltpu.VMEM((CK, CB), jnp.int32),
            pltpu.VMEM((2, CB, HALF), jnp.float32),
            pltpu.VMEM_SHARED((NP, HALF), jnp.float32),
            pltpu.SemaphoreType.DMA((2,)),
        ],
    )


def _segsum_body(x2_hbm, idx2_hbm, dst_hbm, out_hbm,
                 idx_v, dst_v, rows_v, acc, sems):
    c = lax.axis_index("c")
    s = lax.axis_index("s")

    pltpu.sync_copy(idx2_hbm.at[c, s], idx_v)
    pltpu.sync_copy(dst_hbm.at[s], dst_v)

    def _zero_row(i, carry):
        for j in range(HALF // 16):
            rows_v[0, i, 16 * j:16 * (j + 1)] = jnp.zeros((16,), jnp.float32)
        return carry
    lax.fori_loop(0, CB, _zero_row, 0)
    for off in range(0, NR, CB):
        n = min(CB, NR - off)
        pltpu.sync_copy(rows_v.at[0, pl.ds(0, n)],
                        acc.at[pl.ds(s * NR + off, n)])
    plsc.subcore_barrier()

    pltpu.async_copy(x2_hbm.at[idx_v.at[pl.ds(0, CB)]], rows_v.at[0], sems.at[0])

    def _chunk(k, carry):
        b = k % 2

        @pl.when(k + 1 < CK)
        def _():
            pltpu.async_copy(x2_hbm.at[idx_v.at[pl.ds((k + 1) * CB, CB)]],
                             rows_v.at[1 - b], sems.at[1 - b])

        pltpu.make_async_copy(x2_hbm.at[idx_v.at[pl.ds(k * CB, CB)]],
                              rows_v.at[b], sems.at[b]).wait()
        pltpu.sync_copy(rows_v.at[b], acc.at[dst_v.at[k]], add=True)
        return carry

    lax.fori_loop(0, CK, _chunk, 0)
    plsc.subcore_barrier()

    pltpu.sync_copy(acc.at[pl.ds(s * NR, NR)], out_hbm.at[c, pl.ds(s * NR, NR)])


def _mlp_body(add_residual, fuse_head, x_ref, agg_ref, Wa_ref, ba_ref,
              Wb_ref, bb_ref, g_ref, be_ref, *rest):
    if fuse_head:
        Wh1_ref, bh1_ref, Wh2_ref, bh2_ref, out_ref = rest
    else:
        (out_ref,) = rest
    xb = x_ref[...]
    h0 = xb + jnp.concatenate([agg_ref[0], agg_ref[1]], axis=-1)
    h = jnp.dot(h0, Wa_ref[...], preferred_element_type=jnp.float32)
    h = jnp.maximum(h + ba_ref[...], 0.0)
    h = jnp.dot(h, Wb_ref[...], preferred_element_type=jnp.float32) + bb_ref[...]
    mu = jnp.mean(h, axis=-1, keepdims=True)
    var = jnp.mean((h - mu) ** 2, axis=-1, keepdims=True)
    h = (h - mu) / jnp.sqrt(var + 1e-5) * g_ref[...] + be_ref[...]
    h = jnp.maximum(h, 0.0)
    if add_residual:
        h = h + xb
    if fuse_head:
        h = jnp.maximum(
            jnp.dot(h, Wh1_ref[...], preferred_element_type=jnp.float32)
            + bh1_ref[...], 0.0)
        h = jnp.dot(h, Wh2_ref[...], preferred_element_type=jnp.float32) \
            + bh2_ref[...]
    out_ref[...] = h


_TILE = 1000


def _mlp_call(x, agg, weights, add_residual, fuse_head):
    w_specs = []
    for w in weights:
        if w.ndim == 1:
            w = w.reshape(1, -1)
        w_specs.append((w, pl.BlockSpec(w.shape, lambda i: (0, 0))))
    return pl.pallas_call(
        functools.partial(_mlp_body, add_residual, fuse_head),
        grid=(N // _TILE,),
        in_specs=[
            pl.BlockSpec((_TILE, D), lambda i: (i, 0)),
            pl.BlockSpec((NC, _TILE, HALF), lambda i: (0, i, 0)),
        ] + [spec for _, spec in w_specs],
        out_specs=pl.BlockSpec((_TILE, D), lambda i: (i, 0)),
        out_shape=jax.ShapeDtypeStruct((N, D), jnp.float32),
    )(x, agg, *[w for w, _ in w_specs])


def kernel(x, edge_index, W0a, b0a, W0b, b0b, g0, be0, W1a, b1a, W1b, b1b,
           g1, be1, W2a, b2a, W2b, b2b, g2, be2, Wh1, bh1, Wh2, bh2):
    src = edge_index[0]
    dst = edge_index[1]
    idx2 = jnp.stack([src * 2, src * 2 + 1]).reshape(NC, NS, ET)
    dstr = dst.reshape(NS, CK, CB)

    def seg(h):
        return _get_segsum_call()(h.reshape(2 * N, HALF), idx2, dstr)

    h = _mlp_call(x, seg(x), (W0a, b0a, W0b, b0b, g0, be0), False, False)
    h = _mlp_call(h, seg(h), (W1a, b1a, W1b, b1b, g1, be1), True, False)
    out = _mlp_call(h, seg(h),
                    (W2a, b2a, W2b, b2b, g2, be2, Wh1, bh1, Wh2, bh2),
                    True, True)
    return out


# speedup vs baseline: 3.1953x; 1.1759x over previous
import functools

import jax
import jax.numpy as jnp
from jax import lax
from jax.experimental import pallas as pl
from jax.experimental.pallas import tpu as pltpu
from jax.experimental.pallas import tpu_sc as plsc

N = 10000
E = 160000
D = 256
HALF = D // 2

NC = 2
NS = 16
ET = E // NS
CB = 80
ETP = 10000
CK = ETP // CB
NP = 10000
NRA = 632
NRB = NP - (NS - 1) * NRA

@functools.cache
def _get_segsum_call():
    mesh = plsc.VectorSubcoreMesh(core_axis_name="c", subcore_axis_name="s",
                                  num_cores=NC, num_subcores=NS)
    return pl.kernel(
        _segsum_body,
        out_type=jax.ShapeDtypeStruct((NC, NP, HALF), jnp.float32),
        mesh=mesh,
        scratch_types=[
            pltpu.VMEM((ETP,), jnp.int32),
            pltpu.VMEM((ETP,), jnp.int32),
            pltpu.VMEM((3, CB, HALF), jnp.float32),
            pltpu.VMEM_SHARED((NP, HALF), jnp.float32),
            pltpu.SemaphoreType.DMA((3,)),
        ],
    )


def _segsum_body(x2_hbm, idx2_hbm, dst_hbm, out_hbm,
                 idx_v, dst_v, rows_v, acc, sems):
    c = lax.axis_index("c")
    s = lax.axis_index("s")

    pltpu.sync_copy(idx2_hbm.at[c, s], idx_v)
    pltpu.sync_copy(dst_hbm.at[s], dst_v)

    nrs = jnp.where(s < NS - 1, NRA, NRB)
    start = s * NRA

    def _zero_row(i, carry):
        for j in range(HALF // 16):
            rows_v[0, i, 16 * j:16 * (j + 1)] = jnp.zeros((16,), jnp.float32)
        return carry
    lax.fori_loop(0, CB, _zero_row, 0)

    def _ranged_copy(body):
        for off in range(0, NRA, CB):
            @pl.when(off + CB <= nrs)
            def _():
                body(off, CB)
        @pl.when(s < NS - 1)
        def _():
            body((NRA // CB) * CB, NRA % CB)
        @pl.when(s == NS - 1)
        def _():
            body((NRB // CB) * CB, NRB % CB)

    _ranged_copy(lambda off, n: pltpu.sync_copy(
        rows_v.at[0, pl.ds(0, n)], acc.at[pl.ds(start + off, n)]))
    plsc.subcore_barrier()

    for p in range(2):
        pltpu.async_copy(x2_hbm.at[idx_v.at[pl.ds(p * CB, CB)]],
                         rows_v.at[p], sems.at[p])

    def _chunk(k, carry):
        @pl.when(k + 2 < CK)
        def _():
            pltpu.async_copy(x2_hbm.at[idx_v.at[pl.ds((k + 2) * CB, CB)]],
                             rows_v.at[(k + 2) % 3], sems.at[(k + 2) % 3])

        pltpu.make_async_copy(x2_hbm.at[idx_v.at[pl.ds(k * CB, CB)]],
                              rows_v.at[k % 3], sems.at[k % 3]).wait()
        pltpu.sync_copy(rows_v.at[k % 3], acc.at[dst_v.at[pl.ds(k * CB, CB)]],
                        add=True)
        return carry

    lax.fori_loop(0, CK, _chunk, 0)
    plsc.subcore_barrier()

    _ranged_copy(lambda off, n: pltpu.sync_copy(
        acc.at[pl.ds(start + off, n)], out_hbm.at[c, pl.ds(start + off, n)]))


def _mlp_body(add_residual, fuse_head, x_ref, agg_ref, Wa_ref, ba_ref,
              Wb_ref, bb_ref, g_ref, be_ref, *rest):
    if fuse_head:
        Wh1_ref, bh1_ref, Wh2_ref, bh2_ref, out_ref = rest
    else:
        (out_ref,) = rest
    xb = x_ref[...]
    h0 = xb + jnp.concatenate([agg_ref[0], agg_ref[1]], axis=-1)
    h = jnp.dot(h0, Wa_ref[...], preferred_element_type=jnp.float32)
    h = jnp.maximum(h + ba_ref[...], 0.0)
    h = jnp.dot(h, Wb_ref[...], preferred_element_type=jnp.float32) + bb_ref[...]
    mu = jnp.mean(h, axis=-1, keepdims=True)
    var = jnp.mean((h - mu) ** 2, axis=-1, keepdims=True)
    h = (h - mu) / jnp.sqrt(var + 1e-5) * g_ref[...] + be_ref[...]
    h = jnp.maximum(h, 0.0)
    if add_residual:
        h = h + xb
    if fuse_head:
        h = jnp.maximum(
            jnp.dot(h, Wh1_ref[...], preferred_element_type=jnp.float32)
            + bh1_ref[...], 0.0)
        h = jnp.dot(h, Wh2_ref[...], preferred_element_type=jnp.float32) \
            + bh2_ref[...]
    out_ref[...] = h


_TILE = 1000


def _mlp_call(x, agg, weights, add_residual, fuse_head):
    w_specs = []
    for w in weights:
        if w.ndim == 1:
            w = w.reshape(1, -1)
        w_specs.append((w, pl.BlockSpec(w.shape, lambda i: (0, 0))))
    return pl.pallas_call(
        functools.partial(_mlp_body, add_residual, fuse_head),
        grid=(N // _TILE,),
        in_specs=[
            pl.BlockSpec((_TILE, D), lambda i: (i, 0)),
            pl.BlockSpec((NC, _TILE, HALF), lambda i: (0, i, 0)),
        ] + [spec for _, spec in w_specs],
        out_specs=pl.BlockSpec((_TILE, D), lambda i: (i, 0)),
        out_shape=jax.ShapeDtypeStruct((N, D), jnp.float32),
    )(x, agg, *[w for w, _ in w_specs])


def kernel(x, edge_index, W0a, b0a, W0b, b0b, g0, be0, W1a, b1a, W1b, b1b,
           g1, be1, W2a, b2a, W2b, b2b, g2, be2, Wh1, bh1, Wh2, bh2):
    src = edge_index[0]
    dst = edge_index[1]
    idx2 = jnp.stack([src * 2, src * 2 + 1]).reshape(NC, NS, ET)
    dstr = dst.reshape(NS, ET)

    def seg(h):
        return _get_segsum_call()(h.reshape(2 * N, HALF), idx2, dstr)

    h = _mlp_call(x, seg(x), (W0a, b0a, W0b, b0b, g0, be0), False, False)
    h = _mlp_call(h, seg(h), (W1a, b1a, W1b, b1b, g1, be1), True, False)
    out = _mlp_call(h, seg(h),
                    (W2a, b2a, W2b, b2b, g2, be2, Wh1, bh1, Wh2, bh2),
                    True, True)
    return out
